# baseline (speedup 1.0000x reference)
import functools

import jax
import jax.numpy as jnp
from jax import lax
from jax.experimental import pallas as pl
from jax.experimental.pallas import tpu as pltpu
from jax.experimental.pallas import tpu_sc as plsc

BATCH = 16384
D = 128
NC = 2
NS = 16
NW = NC * NS
PER_W = BATCH // NW
CHUNK = 128
NCHUNK = PER_W // CHUNK
L = 16


def _mf_body(u_idx_hbm, i_idx_hbm, gm_hbm, ue_hbm, ie_hbm, ub_hbm, ib_hbm,
             out_hbm,
             u_idx_v, i_idx_v, u_r0, u_r1, i_r0, i_r1, ub_v, ib_v, gm_v, out_v,
             sem_a, sem_b, sem_bias):
    wid = lax.axis_index("s") * NC + lax.axis_index("c")
    base = wid * PER_W

    pltpu.sync_copy(u_idx_hbm.at[pl.ds(base, PER_W)], u_idx_v)
    pltpu.sync_copy(i_idx_hbm.at[pl.ds(base, PER_W)], i_idx_v)
    pltpu.sync_copy(gm_hbm, gm_v)

    u_bufs = (u_r0, u_r1)
    i_bufs = (i_r0, i_r1)
    sems = (sem_a, sem_b)

    def start_chunk(c):
        b = c % 2
        s = pl.ds(c * CHUNK, CHUNK)
        hu = pltpu.async_copy(ue_hbm.at[u_idx_v.at[s]], u_bufs[b], sems[b])
        hi = pltpu.async_copy(ie_hbm.at[i_idx_v.at[s]], i_bufs[b], sems[b])
        return hu, hi

    handles = [start_chunk(0), start_chunk(1)]
    bias_handles = []
    for c in range(NCHUNK):
        s = pl.ds(c * CHUNK, CHUNK)
        bias_handles.append(
            pltpu.async_copy(ub_hbm.at[u_idx_v.at[s]], ub_v.at[s], sem_bias))
        bias_handles.append(
            pltpu.async_copy(ib_hbm.at[i_idx_v.at[s]], ib_v.at[s], sem_bias))

    lane = lax.iota(jnp.int32, L)
    for c in range(NCHUNK):
        hu, hi = handles[c]
        hu.wait()
        hi.wait()
        u_ref = u_bufs[c % 2]
        i_ref = i_bufs[c % 2]

        def grp_body(g, carry, c=c, u_ref=u_ref, i_ref=i_ref):
            rows = g * L + lane
            accs = [jnp.zeros((L,), jnp.float32) for _ in range(4)]
            for d in range(D):
                col = (lane + d) & (D - 1)
                u = plsc.load_gather(u_ref, [rows, col])
                v = plsc.load_gather(i_ref, [rows, col])
                accs[d % 4] = accs[d % 4] + u * v
            acc = (accs[0] + accs[1]) + (accs[2] + accs[3])
            out_v[pl.ds(c * CHUNK + g * L, L)] = acc
            return carry

        lax.fori_loop(0, CHUNK // L, grp_body, 0)
        if c + 2 < NCHUNK:
            handles.append(start_chunk(c + 2))

    for h in bias_handles:
        h.wait()

    gm = gm_v[pl.ds(0, L)]

    def grp_body(g, carry):
        s = pl.ds(g * L, L)
        out_v[s] = out_v[s] + ub_v[s] + ib_v[s] + gm
        return carry

    lax.fori_loop(0, PER_W // L, grp_body, 0, unroll=2)

    pltpu.sync_copy(out_v, out_hbm.at[pl.ds(base, PER_W)])


@jax.jit
def _mf_kernel(u_idx, i_idx, gm8, ue, ie, ub, ib):
    mesh = plsc.VectorSubcoreMesh(core_axis_name="c", subcore_axis_name="s",
                                  num_cores=NC, num_subcores=NS)
    return pl.kernel(
        _mf_body,
        out_type=jax.ShapeDtypeStruct((BATCH,), jnp.float32),
        mesh=mesh,
        scratch_types=[
            pltpu.VMEM((PER_W,), jnp.int32),
            pltpu.VMEM((PER_W,), jnp.int32),
            pltpu.VMEM((CHUNK, D), jnp.float32),
            pltpu.VMEM((CHUNK, D), jnp.float32),
            pltpu.VMEM((CHUNK, D), jnp.float32),
            pltpu.VMEM((CHUNK, D), jnp.float32),
            pltpu.VMEM((PER_W,), jnp.float32),
            pltpu.VMEM((PER_W,), jnp.float32),
            pltpu.VMEM((L,), jnp.float32),
            pltpu.VMEM((PER_W,), jnp.float32),
            pltpu.SemaphoreType.DMA,
            pltpu.SemaphoreType.DMA,
            pltpu.SemaphoreType.DMA,
        ],
        compiler_params=pltpu.CompilerParams(needs_layout_passes=False),
    )(u_idx, i_idx, gm8, ue, ie, ub, ib)


def kernel(user_indices, item_indeices, global_mean, user_emb, item_emb,
           user_bias, item_bias):
    u_idx = user_indices.astype(jnp.int32)
    i_idx = item_indeices.astype(jnp.int32)
    gm8 = jnp.broadcast_to(global_mean.astype(jnp.float32), (16,))
    ub = user_bias.reshape(-1)
    ib = item_bias.reshape(-1)
    return _mf_kernel(u_idx, i_idx, gm8, user_emb, item_emb, ub, ib)


# speedup vs baseline: 1.1477x; 1.1477x over previous
import functools

import jax
import jax.numpy as jnp
from jax import lax
from jax.experimental import pallas as pl
from jax.experimental.pallas import tpu as pltpu
from jax.experimental.pallas import tpu_sc as plsc

BATCH = 16384
D = 128
NC = 2
NS = 16
NW = NC * NS
PER_W = BATCH // NW
CHUNK = 128
NCHUNK = PER_W // CHUNK
L = 16


def _mf_body(u_idx_hbm, i_idx_hbm, gm_hbm, ue_hbm, ie_hbm, ub_hbm, ib_hbm,
             out_hbm,
             u_idx_v, i_idx_v, u_r0, u_r1, i_r0, i_r1, ub_v, ib_v, gm_v, out_v,
             sem_a, sem_b, sem_bias):
    wid = lax.axis_index("s") * NC + lax.axis_index("c")
    base = wid * PER_W

    pltpu.sync_copy(u_idx_hbm.at[pl.ds(base, PER_W)], u_idx_v)
    pltpu.sync_copy(i_idx_hbm.at[pl.ds(base, PER_W)], i_idx_v)
    pltpu.sync_copy(gm_hbm, gm_v)

    u_bufs = (u_r0, u_r1)
    i_bufs = (i_r0, i_r1)
    sems = (sem_a, sem_b)

    def start_chunk(c):
        b = c % 2
        s = pl.ds(c * CHUNK, CHUNK)
        hu = pltpu.async_copy(ue_hbm.at[u_idx_v.at[s]], u_bufs[b], sems[b])
        hi = pltpu.async_copy(ie_hbm.at[i_idx_v.at[s]], i_bufs[b], sems[b])
        return hu, hi

    handles = [start_chunk(0), start_chunk(1)]
    bias_handles = []
    for c in range(NCHUNK):
        s = pl.ds(c * CHUNK, CHUNK)
        bias_handles.append(
            pltpu.async_copy(ub_hbm.at[u_idx_v.at[s]], ub_v.at[s], sem_bias))
        bias_handles.append(
            pltpu.async_copy(ib_hbm.at[i_idx_v.at[s]], ib_v.at[s], sem_bias))

    lane = lax.iota(jnp.int32, L)
    for c in range(NCHUNK):
        hu, hi = handles[c]
        hu.wait()
        hi.wait()
        u_ref = u_bufs[c % 2]
        i_ref = i_bufs[c % 2]

        def grp_body(g, carry, c=c, u_ref=u_ref, i_ref=i_ref):
            rows = g * L + lane
            accs = [jnp.zeros((L,), jnp.float32) for _ in range(2)]
            for d in range(D):
                col = (lane * 9 + d) & (D - 1)
                u = plsc.load_gather(u_ref, [rows, col])
                v = plsc.load_gather(i_ref, [rows, col])
                accs[d % 2] = accs[d % 2] + u * v
            acc = accs[0] + accs[1]
            out_v[pl.ds(c * CHUNK + g * L, L)] = acc
            return carry

        lax.fori_loop(0, CHUNK // L, grp_body, 0)
        if c + 2 < NCHUNK:
            handles.append(start_chunk(c + 2))

    for h in bias_handles:
        h.wait()

    gm = gm_v[pl.ds(0, L)]

    def grp_body(g, carry):
        s = pl.ds(g * L, L)
        out_v[s] = out_v[s] + ub_v[s] + ib_v[s] + gm
        return carry

    lax.fori_loop(0, PER_W // L, grp_body, 0, unroll=2)

    pltpu.sync_copy(out_v, out_hbm.at[pl.ds(base, PER_W)])


@jax.jit
def _mf_kernel(u_idx, i_idx, gm8, ue, ie, ub, ib):
    mesh = plsc.VectorSubcoreMesh(core_axis_name="c", subcore_axis_name="s",
                                  num_cores=NC, num_subcores=NS)
    return pl.kernel(
        _mf_body,
        out_type=jax.ShapeDtypeStruct((BATCH,), jnp.float32),
        mesh=mesh,
        scratch_types=[
            pltpu.VMEM((PER_W,), jnp.int32),
            pltpu.VMEM((PER_W,), jnp.int32),
            pltpu.VMEM((CHUNK, D), jnp.float32),
            pltpu.VMEM((CHUNK, D), jnp.float32),
            pltpu.VMEM((CHUNK, D), jnp.float32),
            pltpu.VMEM((CHUNK, D), jnp.float32),
            pltpu.VMEM((PER_W,), jnp.float32),
            pltpu.VMEM((PER_W,), jnp.float32),
            pltpu.VMEM((L,), jnp.float32),
            pltpu.VMEM((PER_W,), jnp.float32),
            pltpu.SemaphoreType.DMA,
            pltpu.SemaphoreType.DMA,
            pltpu.SemaphoreType.DMA,
        ],
        compiler_params=pltpu.CompilerParams(needs_layout_passes=False),
    )(u_idx, i_idx, gm8, ue, ie, ub, ib)


def kernel(user_indices, item_indeices, global_mean, user_emb, item_emb,
           user_bias, item_bias):
    u_idx = user_indices.astype(jnp.int32)
    i_idx = item_indeices.astype(jnp.int32)
    gm8 = jnp.broadcast_to(global_mean.astype(jnp.float32), (16,))
    ub = user_bias.reshape(-1)
    ib = item_bias.reshape(-1)
    return _mf_kernel(u_idx, i_idx, gm8, user_emb, item_emb, ub, ib)


# speedup vs baseline: 1.5228x; 1.3268x over previous
import functools

import jax
import jax.numpy as jnp
from jax import lax
from jax.experimental import pallas as pl
from jax.experimental.pallas import tpu as pltpu
from jax.experimental.pallas import tpu_sc as plsc

BATCH = 16384
D = 128
NC = 2
NS = 16
NW = NC * NS
PER_W = BATCH // NW
CHUNK = 128
NCHUNK = PER_W // CHUNK
L = 16


def _mf_body(u_idx_hbm, i_idx_hbm, gm_hbm, ue_hbm, ie_hbm, ub_hbm, ib_hbm,
             out_hbm,
             u_idx_v, i_idx_v, u_r0, u_r1, i_r0, i_r1, ub_v, ib_v, gm_v, out_v,
             partials, sem_a, sem_b, sem_bias):
    wid = lax.axis_index("s") * NC + lax.axis_index("c")
    base = wid * PER_W

    pltpu.sync_copy(u_idx_hbm.at[pl.ds(base, PER_W)], u_idx_v)
    pltpu.sync_copy(i_idx_hbm.at[pl.ds(base, PER_W)], i_idx_v)
    pltpu.sync_copy(gm_hbm, gm_v)

    u_bufs = (u_r0, u_r1)
    i_bufs = (i_r0, i_r1)
    sems = (sem_a, sem_b)

    def start_chunk(c):
        b = c % 2
        s = pl.ds(c * CHUNK, CHUNK)
        hu = pltpu.async_copy(ue_hbm.at[u_idx_v.at[s]], u_bufs[b], sems[b])
        hi = pltpu.async_copy(ie_hbm.at[i_idx_v.at[s]], i_bufs[b], sems[b])
        return hu, hi

    handles = [start_chunk(0), start_chunk(1)]
    bias_handles = []
    for c in range(NCHUNK):
        s = pl.ds(c * CHUNK, CHUNK)
        bias_handles.append(
            pltpu.async_copy(ub_hbm.at[u_idx_v.at[s]], ub_v.at[s], sem_bias))
        bias_handles.append(
            pltpu.async_copy(ib_hbm.at[i_idx_v.at[s]], ib_v.at[s], sem_bias))

    lane = lax.iota(jnp.int32, L)
    for c in range(NCHUNK):
        hu, hi = handles[c]
        hu.wait()
        hi.wait()
        u_ref = u_bufs[c % 2]
        i_ref = i_bufs[c % 2]

        def grp_body(g, carry, c=c, u_ref=u_ref, i_ref=i_ref):
            def pair_regs(j):
                p = g * L + j
                return ([u_ref[p, pl.ds(k * L, L)] for k in range(D // L)],
                        [i_ref[p, pl.ds(k * L, L)] for k in range(D // L)])

            us, vs = pair_regs(0)
            for j in range(L):
                nxt = pair_regs(j + 1) if j + 1 < L else None
                prods = [u * v for u, v in zip(us, vs)]
                s0 = (prods[0] + prods[1]) + (prods[2] + prods[3])
                s1 = (prods[4] + prods[5]) + (prods[6] + prods[7])
                partials[j] = s0 + s1
                if nxt is not None:
                    us, vs = nxt
            tot = jnp.zeros((L,), jnp.float32)
            for l in range(L):
                col = (lane + l) & (L - 1)
                tot = tot + plsc.load_gather(partials, [lane, col])
            out_v[pl.ds(c * CHUNK + g * L, L)] = tot
            return carry

        lax.fori_loop(0, CHUNK // L, grp_body, 0)
        if c + 2 < NCHUNK:
            handles.append(start_chunk(c + 2))

    for h in bias_handles:
        h.wait()

    gm = gm_v[pl.ds(0, L)]

    def grp_body(g, carry):
        s = pl.ds(g * L, L)
        out_v[s] = out_v[s] + ub_v[s] + ib_v[s] + gm
        return carry

    lax.fori_loop(0, PER_W // L, grp_body, 0, unroll=2)

    pltpu.sync_copy(out_v, out_hbm.at[pl.ds(base, PER_W)])


@jax.jit
def _mf_kernel(u_idx, i_idx, gm8, ue, ie, ub, ib):
    mesh = plsc.VectorSubcoreMesh(core_axis_name="c", subcore_axis_name="s",
                                  num_cores=NC, num_subcores=NS)
    return pl.kernel(
        _mf_body,
        out_type=jax.ShapeDtypeStruct((BATCH,), jnp.float32),
        mesh=mesh,
        scratch_types=[
            pltpu.VMEM((PER_W,), jnp.int32),
            pltpu.VMEM((PER_W,), jnp.int32),
            pltpu.VMEM((CHUNK, D), jnp.float32),
            pltpu.VMEM((CHUNK, D), jnp.float32),
            pltpu.VMEM((CHUNK, D), jnp.float32),
            pltpu.VMEM((CHUNK, D), jnp.float32),
            pltpu.VMEM((PER_W,), jnp.float32),
            pltpu.VMEM((PER_W,), jnp.float32),
            pltpu.VMEM((L,), jnp.float32),
            pltpu.VMEM((PER_W,), jnp.float32),
            pltpu.VMEM((L, L), jnp.float32),
            pltpu.SemaphoreType.DMA,
            pltpu.SemaphoreType.DMA,
            pltpu.SemaphoreType.DMA,
        ],
        compiler_params=pltpu.CompilerParams(needs_layout_passes=False),
    )(u_idx, i_idx, gm8, ue, ie, ub, ib)


def kernel(user_indices, item_indeices, global_mean, user_emb, item_emb,
           user_bias, item_bias):
    u_idx = user_indices.astype(jnp.int32)
    i_idx = item_indeices.astype(jnp.int32)
    gm8 = jnp.broadcast_to(global_mean.astype(jnp.float32), (16,))
    ub = user_bias.reshape(-1)
    ib = item_bias.reshape(-1)
    return _mf_kernel(u_idx, i_idx, gm8, user_emb, item_emb, ub, ib)
